# baseline (speedup 1.0000x reference)
import functools

import jax
import jax.numpy as jnp
from jax import lax
from jax.experimental import pallas as pl
from jax.experimental.pallas import tpu as pltpu
from jax.experimental.pallas import tpu_sc as plsc

C = 3
T = 64
TS = T // 4
ROW = 224 * 224
HALF = ROW // 2
NW = 32
NCHUNK = 2 * C * TS
PER_W = NCHUNK // NW


def _slow_gather(frames_flat):
    mesh = plsc.VectorSubcoreMesh(core_axis_name="c", subcore_axis_name="s")

    @functools.partial(
        pl.kernel,
        out_type=jax.ShapeDtypeStruct((C * TS * ROW,), jnp.float32),
        mesh=mesh,
        scratch_types=[pltpu.VMEM((HALF,), jnp.float32)],
    )
    def k(src, dst, buf):
        w = lax.axis_index("s") * 2 + lax.axis_index("c")
        for i in range(PER_W):
            chunk = w + i * NW
            out_row = chunk // 2
            half = chunk % 2
            ch = out_row // TS
            j = out_row % TS
            t = (j * 21) // 5
            src_off = (ch * T + t) * ROW + half * HALF
            dst_off = out_row * ROW + half * HALF
            pltpu.sync_copy(src.at[pl.ds(src_off, HALF)], buf)
            pltpu.sync_copy(buf, dst.at[pl.ds(dst_off, HALF)])

    return k(frames_flat)


def kernel(frames):
    slow = _slow_gather(frames.reshape(-1)).reshape(C, TS, 224, 224)
    return (slow, frames)


# speedup vs baseline: 2.0455x; 2.0455x over previous
import functools

import jax
import jax.numpy as jnp
from jax import lax
from jax.experimental import pallas as pl
from jax.experimental.pallas import tpu as pltpu
from jax.experimental.pallas import tpu_sc as plsc

C = 3
T = 64
TS = T // 4
H = 224
W = 224
NW = 32
UNITS = C * T
PER_W = UNITS // NW


def _pack(frames):
    mesh = plsc.VectorSubcoreMesh(core_axis_name="c", subcore_axis_name="s")

    @functools.partial(
        pl.kernel,
        out_type=(
            jax.ShapeDtypeStruct((C, TS, H, W), jnp.float32),
            jax.ShapeDtypeStruct((C, T, H, W), jnp.float32),
        ),
        mesh=mesh,
        scratch_types=[pltpu.VMEM((H, W), jnp.float32)],
        compiler_params=pltpu.CompilerParams(use_tc_tiling_on_sc=True),
    )
    def k(src, slow, fast, buf):
        wid = lax.axis_index("s") * 2 + lax.axis_index("c")
        for i in range(PER_W):
            u = wid + i * NW
            c = u // T
            t = u % T
            pltpu.sync_copy(src.at[c, t], buf)
            pltpu.sync_copy(buf, fast.at[c, t])
            j = (5 * t + 20) // 21
            sel = (21 * j) // 5 == t
            @pl.when(sel)
            def _():
                pltpu.sync_copy(buf, slow.at[c, j])

    return k(frames)


def kernel(frames):
    slow, fast = _pack(frames)
    return (slow, fast)


# speedup vs baseline: 2.1969x; 1.0740x over previous
import functools

import jax
import jax.numpy as jnp
from jax import lax
from jax.experimental import pallas as pl
from jax.experimental.pallas import tpu as pltpu
from jax.experimental.pallas import tpu_sc as plsc

C = 3
T = 64
TS = T // 4
H = 224
W = 224
CH = H // 2
NW = 32
CHUNKS = C * T * 2
PER_W = CHUNKS // NW
NB = 4


def _pack(frames):
    mesh = plsc.VectorSubcoreMesh(core_axis_name="c", subcore_axis_name="s")

    @functools.partial(
        pl.kernel,
        out_type=(
            jax.ShapeDtypeStruct((C, TS, H, W), jnp.float32),
            jax.ShapeDtypeStruct((C, T, H, W), jnp.float32),
        ),
        mesh=mesh,
        scratch_types=[
            [pltpu.VMEM((CH, W), jnp.float32) for _ in range(NB)],
            [pltpu.SemaphoreType.DMA for _ in range(NB)],
            [pltpu.SemaphoreType.DMA for _ in range(NB)],
        ],
        compiler_params=pltpu.CompilerParams(use_tc_tiling_on_sc=True),
    )
    def k(src, slow, fast, bufs, in_sems, out_sems):
        wid = lax.axis_index("s") * 2 + lax.axis_index("c")
        base = wid * PER_W

        def coords(i):
            m = base + i
            u = m // 2
            half = m % 2
            c = u // T
            t = u % T
            j = (5 * t + 20) // 21
            sel = (21 * j) // 5 == t
            return c, t, half, j, sel

        def src_sl(c, t, half):
            return src.at[c, t, pl.ds(half * CH, CH), :]

        def fast_sl(c, t, half):
            return fast.at[c, t, pl.ds(half * CH, CH), :]

        def slow_sl(c, j, half):
            return slow.at[c, j, pl.ds(half * CH, CH), :]

        def wait_out(i):
            c, t, half, j, sel = coords(i)
            b = i % NB
            pltpu.make_async_copy(bufs[b], fast_sl(c, t, half), out_sems[b]).wait()

            @pl.when(sel)
            def _():
                pltpu.make_async_copy(bufs[b], slow_sl(c, j, half), out_sems[b]).wait()

        for p in range(NB - 1):
            c, t, half, _, _ = coords(p)
            pltpu.async_copy(src_sl(c, t, half), bufs[p], in_sems[p])

        for i in range(PER_W):
            b = i % NB
            c, t, half, j, sel = coords(i)
            pltpu.make_async_copy(src_sl(c, t, half), bufs[b], in_sems[b]).wait()
            pltpu.async_copy(bufs[b], fast_sl(c, t, half), out_sems[b])

            @pl.when(sel)
            def _():
                pltpu.async_copy(bufs[b], slow_sl(c, j, half), out_sems[b])

            nxt = i + NB - 1
            if nxt < PER_W:
                if i >= 1:
                    wait_out(i - 1)
                cn, tn, hn, _, _ = coords(nxt)
                pltpu.async_copy(src_sl(cn, tn, hn), bufs[nxt % NB], in_sems[nxt % NB])

        for i in range(PER_W - NB, PER_W):
            wait_out(i)

    return k(frames)


def kernel(frames):
    slow, fast = _pack(frames)
    return (slow, fast)
